# baseline (speedup 1.0000x reference)
import functools

import jax
import jax.numpy as jnp
from jax import lax
from jax.experimental import pallas as pl
from jax.experimental.pallas import tpu as pltpu
from jax.experimental.pallas import tpu_sc as plsc

B = 16384
D = 128
NC, NS = 2, 16
NW = NC * NS
BPW = B // NW
GCH = 128

_mesh = plsc.VectorSubcoreMesh(core_axis_name="c", subcore_axis_name="s")


@functools.partial(
    pl.kernel,
    out_type=[
        jax.ShapeDtypeStruct((B, D), jnp.float32),
        jax.ShapeDtypeStruct((B, D), jnp.float32),
    ],
    mesh=_mesh,
    scratch_types=[
        pltpu.VMEM((BPW,), jnp.int32),
        pltpu.VMEM((BPW, D), jnp.float32),
        pltpu.SemaphoreType.DMA,
    ],
)
def _sc_gather2(uidx_hbm, iidx_hbm, utab_hbm, itab_hbm, u_out, v_out,
                idx_v, rows_v, sem):
    wid = lax.axis_index("s") * NC + lax.axis_index("c")
    base = wid * BPW
    for idx_hbm, tab_hbm, out_hbm in (
        (uidx_hbm, utab_hbm, u_out),
        (iidx_hbm, itab_hbm, v_out),
    ):
        pltpu.sync_copy(idx_hbm.at[pl.ds(base, BPW)], idx_v)
        cps = [
            pltpu.async_copy(
                tab_hbm.at[idx_v.at[pl.ds(j * GCH, GCH)]],
                rows_v.at[pl.ds(j * GCH, GCH)],
                sem,
            )
            for j in range(BPW // GCH)
        ]
        for c in cps:
            c.wait()
        pltpu.sync_copy(rows_v, out_hbm.at[pl.ds(base, BPW)])


TCB = 2048


def _tc_dot_body(u_ref, v_ref, o_ref):
    o_ref[...] = jnp.sum(u_ref[...] * v_ref[...], axis=1, keepdims=True)


_tc_dot = pl.pallas_call(
    _tc_dot_body,
    grid=(B // TCB,),
    in_specs=[
        pl.BlockSpec((TCB, D), lambda i: (i, 0)),
        pl.BlockSpec((TCB, D), lambda i: (i, 0)),
    ],
    out_specs=pl.BlockSpec((TCB, 1), lambda i: (i, 0)),
    out_shape=jax.ShapeDtypeStruct((B, 1), jnp.float32),
)


def kernel(x, user_embedding, item_embedding):
    uidx = x[:, 0].astype(jnp.int32)
    iidx = x[:, 1].astype(jnp.int32)
    u, v = _sc_gather2(uidx, iidx, user_embedding, item_embedding)
    return _tc_dot(u, v).reshape(B)
